# baseline (speedup 1.0000x reference)
import functools

import jax
import jax.numpy as jnp
from jax import lax
from jax.experimental import pallas as pl
from jax.experimental.pallas import tpu as pltpu
from jax.experimental.pallas import tpu_sc as plsc

_B, _C, _HW = 32, 16, 128 * 128
_K = 500
_KP = 512
_NCHUNK = _KP // 16
_UNROLL = 4


def _sc_body(out_hbm, idx_hbm, mask_hbm, tgt_hbm, part_hbm,
             idx_v, mask_v, tgt_v, plane0, plane1, out_v, sem0, sem1):
    b = lax.axis_index("s") * 2 + lax.axis_index("c")

    pltpu.sync_copy(idx_hbm.at[pl.ds(b * _KP, _KP)], idx_v)
    pltpu.sync_copy(mask_hbm.at[pl.ds(b * _KP, _KP)], mask_v)
    pltpu.sync_copy(tgt_hbm.at[b], tgt_v)

    planes = (plane0, plane1)
    sems = (sem0, sem1)
    copies = [None, None]
    copies[0] = pltpu.async_copy(out_hbm.at[b, 0], plane0, sem0)

    acc = jnp.zeros((16,), jnp.float32)
    for c in range(_C):
        buf = planes[c % 2]
        copies[c % 2].wait()
        if c + 1 < _C:
            copies[(c + 1) % 2] = pltpu.async_copy(
                out_hbm.at[b, c + 1], planes[(c + 1) % 2], sems[(c + 1) % 2])

        def chunk(jj, a, buf=buf, c=c):
            for u in range(_UNROLL):
                j = jj * _UNROLL + u
                i16 = idx_v[pl.ds(j * 16, 16)]
                p = plsc.load_gather(buf, [i16 >> 7, i16 & 127])
                t = tgt_v[c * 4 + j // 8, pl.ds((j % 8) * 16, 16)]
                m = mask_v[pl.ds(j * 16, 16)]
                a = a + jnp.abs(m * p - t)
            return a

        acc = lax.fori_loop(0, _NCHUNK // _UNROLL, chunk, acc)

    msum = lax.fori_loop(
        0, _NCHUNK,
        lambda j, a: a + mask_v[pl.ds(j * 16, 16)],
        jnp.zeros((16,), jnp.float32))

    out_v[pl.ds(0, 16)] = acc
    out_v[pl.ds(16, 16)] = msum
    pltpu.sync_copy(out_v, part_hbm.at[b])


_launch = functools.partial(
    pl.kernel,
    mesh=plsc.VectorSubcoreMesh(core_axis_name="c", subcore_axis_name="s"),
    out_type=jax.ShapeDtypeStruct((_B, 32), jnp.float32),
    scratch_types=[
        pltpu.VMEM((_KP,), jnp.int32),
        pltpu.VMEM((_KP,), jnp.float32),
        pltpu.VMEM((_C * _KP // 128, 128), jnp.float32),
        pltpu.VMEM((128, 128), jnp.float32),
        pltpu.VMEM((128, 128), jnp.float32),
        pltpu.VMEM((32,), jnp.float32),
        pltpu.SemaphoreType.DMA,
        pltpu.SemaphoreType.DMA,
    ],
    compiler_params=pltpu.CompilerParams(needs_layout_passes=False),
)(_sc_body)


@jax.jit
def kernel(output, mask, index, target):
    pad = _KP - _K
    idx_p = jnp.pad(index.astype(jnp.int32), ((0, 0), (0, pad))).reshape(-1)
    mask_f = mask.astype(jnp.float32)
    mask_p = jnp.pad(mask_f, ((0, 0), (0, pad))).reshape(-1)
    tgt_t = jnp.transpose(target * mask_f[:, :, None], (0, 2, 1))
    tgt_p = jnp.pad(tgt_t, ((0, 0), (0, 0), (0, pad))).reshape(
        _B, _C * _KP // 128, 128)
    parts = _launch(output, idx_p, mask_p, tgt_p)
    s = jnp.sum(parts[:, :16])
    m = jnp.sum(parts[:, 16:])
    return s / (_C * m + 0.0001)


# speedup vs baseline: 1.0667x; 1.0667x over previous
import functools

import jax
import jax.numpy as jnp
from jax import lax
from jax.experimental import pallas as pl
from jax.experimental.pallas import tpu as pltpu
from jax.experimental.pallas import tpu_sc as plsc

_B, _C, _HW = 32, 16, 128 * 128
_K = 500
_KP = 512
_NCHUNK = _KP // 16
_UNROLL = 4


def _sc_body(out_hbm, idx_hbm, mask_hbm, tgt_hbm, part_hbm,
             idx_v, mask_v, tgt_v, plane0, plane1, out_v, sem0, sem1):
    b = lax.axis_index("s") * 2 + lax.axis_index("c")

    pltpu.sync_copy(idx_hbm.at[pl.ds(b * _KP, _KP)], idx_v)
    pltpu.sync_copy(mask_hbm.at[pl.ds(b * _KP, _KP)], mask_v)
    pltpu.sync_copy(tgt_hbm.at[b], tgt_v)

    planes = (plane0, plane1)
    sems = (sem0, sem1)
    copies = [None, None]
    copies[0] = pltpu.async_copy(out_hbm.at[b, 0], plane0, sem0)

    acc = jnp.zeros((16,), jnp.float32)
    for c in range(_C):
        buf = planes[c % 2]
        copies[c % 2].wait()
        if c + 1 < _C:
            copies[(c + 1) % 2] = pltpu.async_copy(
                out_hbm.at[b, c + 1], planes[(c + 1) % 2], sems[(c + 1) % 2])

        def chunk(jj, a, buf=buf, c=c):
            for u in range(_UNROLL):
                j = jj * _UNROLL + u
                i16 = idx_v[pl.ds(j * 16, 16)]
                p = plsc.load_gather(buf, [i16 >> 7, i16 & 127])
                t = tgt_v[pl.ds(c * _KP + j * 16, 16)]
                m = mask_v[pl.ds(j * 16, 16)]
                a = a + jnp.abs(m * p - t)
            return a

        acc = lax.fori_loop(0, _NCHUNK // _UNROLL, chunk, acc)

    msum = lax.fori_loop(
        0, _NCHUNK,
        lambda j, a: a + mask_v[pl.ds(j * 16, 16)],
        jnp.zeros((16,), jnp.float32))

    out_v[pl.ds(0, 16)] = acc
    out_v[pl.ds(16, 16)] = msum
    pltpu.sync_copy(out_v, part_hbm.at[b])


_launch = functools.partial(
    pl.kernel,
    mesh=plsc.VectorSubcoreMesh(core_axis_name="c", subcore_axis_name="s"),
    out_type=jax.ShapeDtypeStruct((_B, 32), jnp.float32),
    scratch_types=[
        pltpu.VMEM((_KP,), jnp.int32),
        pltpu.VMEM((_KP,), jnp.float32),
        pltpu.VMEM((_C * _KP,), jnp.float32),
        pltpu.VMEM((128, 128), jnp.float32),
        pltpu.VMEM((128, 128), jnp.float32),
        pltpu.VMEM((32,), jnp.float32),
        pltpu.SemaphoreType.DMA,
        pltpu.SemaphoreType.DMA,
    ],
    compiler_params=pltpu.CompilerParams(needs_layout_passes=False),
)(_sc_body)


@jax.jit
def kernel(output, mask, index, target):
    pad = _KP - _K
    idx_p = jnp.pad(index.astype(jnp.int32), ((0, 0), (0, pad))).reshape(-1)
    mask_f = mask.astype(jnp.float32)
    mask_p = jnp.pad(mask_f, ((0, 0), (0, pad))).reshape(-1)
    tgt_t = jnp.transpose(target * mask_f[:, :, None], (0, 2, 1))
    tgt_p = jnp.pad(tgt_t, ((0, 0), (0, 0), (0, pad))).reshape(_B, _C * _KP)
    parts = _launch(output, idx_p, mask_p, tgt_p)
    s = jnp.sum(parts[:, :16])
    m = jnp.sum(parts[:, 16:])
    return s / (_C * m + 0.0001)


# speedup vs baseline: 1.2454x; 1.1676x over previous
import functools

import jax
import jax.numpy as jnp
from jax import lax
from jax.experimental import pallas as pl
from jax.experimental.pallas import tpu as pltpu
from jax.experimental.pallas import tpu_sc as plsc

_B, _C, _HW = 32, 16, 128 * 128
_K = 500
_KP = 512
_NCHUNK = _KP // 16
_UNROLL = 4


def _sc_body(out_hbm, idx_hbm, mask_hbm, tgt_hbm, part_hbm,
             idx_v, mask_v, tgt_v, idxg_v, pred_v, out_v, sem0):
    b = lax.axis_index("s") * 2 + lax.axis_index("c")

    pltpu.sync_copy(idx_hbm.at[pl.ds(b * _KP, _KP)], idx_v)
    pltpu.sync_copy(mask_hbm.at[pl.ds(b * _KP, _KP)], mask_v)
    pltpu.sync_copy(tgt_hbm.at[b], tgt_v)

    for c in range(_C):
        base = (b * _C + c) * _HW

        def bld(jj, _, c=c, base=base):
            idxg_v[pl.ds(c * _KP + jj * 16, 16)] = idx_v[pl.ds(jj * 16, 16)] + base
            return 0

        lax.fori_loop(0, _NCHUNK, bld, 0)

    copies = []
    for d in range(_C * _KP // 128):
        copies.append(pltpu.async_copy(
            out_hbm.at[idxg_v.at[pl.ds(d * 128, 128)]],
            pred_v.at[pl.ds(d * 128, 128)], sem0))
    for cp in copies:
        cp.wait()

    acc = jnp.zeros((16,), jnp.float32)
    for c in range(_C):
        def chunk(jj, a, c=c):
            for u in range(_UNROLL):
                j = jj * _UNROLL + u
                p = pred_v[pl.ds(c * _KP + j * 16, 16)]
                t = tgt_v[pl.ds(c * _KP + j * 16, 16)]
                m = mask_v[pl.ds(j * 16, 16)]
                a = a + jnp.abs(m * p - t)
            return a

        acc = lax.fori_loop(0, _NCHUNK // _UNROLL, chunk, acc)

    msum = lax.fori_loop(
        0, _NCHUNK,
        lambda j, a: a + mask_v[pl.ds(j * 16, 16)],
        jnp.zeros((16,), jnp.float32))

    out_v[pl.ds(0, 16)] = acc
    out_v[pl.ds(16, 16)] = msum
    pltpu.sync_copy(out_v, part_hbm.at[b])


_launch = functools.partial(
    pl.kernel,
    mesh=plsc.VectorSubcoreMesh(core_axis_name="c", subcore_axis_name="s"),
    out_type=jax.ShapeDtypeStruct((_B, 32), jnp.float32),
    scratch_types=[
        pltpu.VMEM((_KP,), jnp.int32),
        pltpu.VMEM((_KP,), jnp.float32),
        pltpu.VMEM((_C * _KP,), jnp.float32),
        pltpu.VMEM((_C * _KP,), jnp.int32),
        pltpu.VMEM((_C * _KP,), jnp.float32),
        pltpu.VMEM((32,), jnp.float32),
        pltpu.SemaphoreType.DMA,
    ],
    compiler_params=pltpu.CompilerParams(needs_layout_passes=False),
)(_sc_body)


@jax.jit
def kernel(output, mask, index, target):
    pad = _KP - _K
    out_flat = output.reshape(-1)
    idx_p = jnp.pad(index.astype(jnp.int32), ((0, 0), (0, pad))).reshape(-1)
    mask_f = mask.astype(jnp.float32)
    mask_p = jnp.pad(mask_f, ((0, 0), (0, pad))).reshape(-1)
    tgt_t = jnp.transpose(target * mask_f[:, :, None], (0, 2, 1))
    tgt_p = jnp.pad(tgt_t, ((0, 0), (0, 0), (0, pad))).reshape(_B, _C * _KP)
    parts = _launch(out_flat, idx_p, mask_p, tgt_p)
    s = jnp.sum(parts[:, :16])
    m = jnp.sum(parts[:, 16:])
    return s / (_C * m + 0.0001)
